# baseline (speedup 1.0000x reference)
import functools

import jax
import jax.numpy as jnp
from jax import lax
from jax.experimental import pallas as pl
from jax.experimental.pallas import tpu as pltpu
from jax.experimental.pallas import tpu_sc as plsc

_B = 8192
_NW = 32
_BPW = _B // _NW
_G = 16


def _sc_body(n_hbm, tmpl_hbm, out_hbm, nv, tmpl, sem):
    wid = lax.axis_index("s") * 2 + lax.axis_index("c")
    base = wid * _BPW

    pltpu.sync_copy(tmpl_hbm, tmpl)
    pltpu.sync_copy(n_hbm.at[pl.ds(base, _BPW)], nv)

    def issue_group(g, carry):
        @pl.when(g >= 1)
        def _wait_prev_group():
            def drain_one(j, c):
                pltpu.make_async_copy(
                    tmpl.at[pl.ds(0, 100), :], out_hbm.at[0], sem
                ).wait()
                return c

            lax.fori_loop(0, _G, drain_one, 0)

        offs = 99 - nv[pl.ds(g * _G, _G)]
        for k in range(_G):
            pltpu.make_async_copy(
                tmpl.at[pl.ds(offs[k], 100), :],
                out_hbm.at[base + g * _G + k],
                sem,
            ).start()
        return carry

    lax.fori_loop(0, _BPW // _G, issue_group, 0)

    def drain(j, carry):
        pltpu.make_async_copy(
            tmpl.at[pl.ds(0, 100), :], out_hbm.at[0], sem
        ).wait()
        return carry

    lax.fori_loop(0, _G, drain, 0)


_sc_fill = functools.partial(
    pl.kernel,
    out_type=jax.ShapeDtypeStruct((_B, 100, 100), jnp.float32),
    mesh=plsc.VectorSubcoreMesh(core_axis_name="c", subcore_axis_name="s"),
    scratch_types=[
        pltpu.VMEM((_BPW,), jnp.int32),
        pltpu.VMEM((199, 100), jnp.float32),
        pltpu.SemaphoreType.DMA,
    ],
)(_sc_body)


def kernel(tensor_span):
    b = tensor_span.shape[0]
    n = tensor_span[:, 0]
    tmpl = (jnp.arange(199, dtype=jnp.int32)[:, None] < 99).astype(jnp.float32)
    tmpl = jnp.broadcast_to(tmpl, (199, 100))
    return _sc_fill(n, tmpl)
